# baseline (speedup 1.0000x reference)
import functools

import jax
import jax.numpy as jnp
from jax import lax
from jax.experimental import pallas as pl
from jax.experimental.pallas import tpu as pltpu
from jax.experimental.pallas import tpu_sc as plsc

_BATCH = 16384
_DIM = 64
_CHUNK = 128


def _two_tower_sc(u, p, user_emb, prod_emb):
    info = plsc.get_sparse_core_info()
    nw = info.num_cores * info.num_subcores
    b_per_w = _BATCH // nw
    n_chunks = b_per_w // _CHUNK
    mesh = plsc.VectorSubcoreMesh(core_axis_name="c", subcore_axis_name="s")

    @functools.partial(
        pl.kernel,
        mesh=mesh,
        out_type=jax.ShapeDtypeStruct((_BATCH,), jnp.float32),
        compiler_params=pltpu.CompilerParams(use_tc_tiling_on_sc=False),
        scratch_types=[
            pltpu.VMEM((n_chunks, _CHUNK), jnp.int32),
            pltpu.VMEM((n_chunks, _CHUNK), jnp.int32),
            pltpu.VMEM((b_per_w, _DIM), jnp.float32),
            pltpu.VMEM((b_per_w, _DIM), jnp.float32),
            pltpu.VMEM((b_per_w,), jnp.float32),
            pltpu.SemaphoreType.DMA,
        ],
    )
    def tile_task(u_hbm, p_hbm, ue_hbm, pe_hbm, out_hbm,
                  uidx_v, pidx_v, urows_v, prows_v, out_v, sem):
        wid = lax.axis_index("s") * info.num_cores + lax.axis_index("c")
        base = wid * b_per_w

        for j in range(n_chunks):
            pltpu.sync_copy(u_hbm.at[pl.ds(base + j * _CHUNK, _CHUNK)],
                            uidx_v.at[j])
            pltpu.sync_copy(p_hbm.at[pl.ds(base + j * _CHUNK, _CHUNK)],
                            pidx_v.at[j])

        copies = []
        for j in range(n_chunks):
            copies.append(pltpu.async_copy(
                ue_hbm.at[uidx_v.at[j]],
                urows_v.at[pl.ds(j * _CHUNK, _CHUNK)], sem))
            copies.append(pltpu.async_copy(
                pe_hbm.at[pidx_v.at[j]],
                prows_v.at[pl.ds(j * _CHUNK, _CHUNK)], sem))
        for c in copies:
            c.wait()

        iota16 = lax.iota(jnp.int32, 16)
        dnums = lax.GatherDimensionNumbers(
            offset_dims=(), collapsed_slice_dims=(0,), start_index_map=(0,))

        def lane_perm(x, idx):
            return lax.gather(
                x, idx[:, None], dimension_numbers=dnums, slice_sizes=(1,),
                mode=lax.GatherScatterMode.PROMISE_IN_BOUNDS)

        def block(b, carry):
            r0 = b * 16
            acc = jnp.zeros((16,), jnp.float32)
            for i in range(16):
                r = r0 + i
                prod = jnp.zeros((16,), jnp.float32)
                for k in range(_DIM // 16):
                    uv = urows_v[r, pl.ds(k * 16, 16)]
                    pv = prows_v[r, pl.ds(k * 16, 16)]
                    prod = prod + uv * pv
                for sh in (8, 4, 2, 1):
                    prod = prod + lane_perm(prod, iota16 ^ sh)
                acc = jnp.where(iota16 == i, prod, acc)
            out_v[pl.ds(r0, 16)] = 1.0 / (1.0 + jnp.exp(-acc))
            return carry

        lax.fori_loop(0, b_per_w // 16, block, 0)
        pltpu.sync_copy(out_v, out_hbm.at[pl.ds(base, b_per_w)])

    return tile_task(u, p, user_emb, prod_emb)


def kernel(u, p, user_emb, prod_emb):
    return _two_tower_sc(u, p, user_emb, prod_emb)


# speedup vs baseline: 1.0008x; 1.0008x over previous
import functools

import jax
import jax.numpy as jnp
from jax import lax
from jax.experimental import pallas as pl
from jax.experimental.pallas import tpu as pltpu
from jax.experimental.pallas import tpu_sc as plsc

_BATCH = 16384
_DIM = 64
_CHUNK = 128

_INFO = plsc.get_sparse_core_info()
_NW = _INFO.num_cores * _INFO.num_subcores
_BPW = _BATCH // _NW
_NCH = _BPW // _CHUNK


def _worker_id():
    return lax.axis_index("s") * _INFO.num_cores + lax.axis_index("c")


def _mesh():
    return plsc.VectorSubcoreMesh(core_axis_name="c", subcore_axis_name="s")


def _gather_rows(idx_hbm, tab_hbm, rows_hbm, idx_v, rows_v, sem, base):
    for j in range(_NCH):
        pltpu.sync_copy(idx_hbm.at[pl.ds(base + j * _CHUNK, _CHUNK)],
                        idx_v.at[j])
    copies = [
        pltpu.async_copy(tab_hbm.at[idx_v.at[j]],
                         rows_v.at[pl.ds(j * _CHUNK, _CHUNK)], sem)
        for j in range(_NCH)
    ]
    for c in copies:
        c.wait()
    pltpu.sync_copy(rows_v, rows_hbm.at[pl.ds(base, _BPW)])


def _user_gather(u, user_emb):
    @functools.partial(
        pl.kernel,
        mesh=_mesh(),
        out_type=jax.ShapeDtypeStruct((_BATCH, _DIM), jnp.float32),
        compiler_params=pltpu.CompilerParams(use_tc_tiling_on_sc=False),
        scratch_types=[
            pltpu.VMEM((_NCH, _CHUNK), jnp.int32),
            pltpu.VMEM((_BPW, _DIM), jnp.float32),
            pltpu.SemaphoreType.DMA,
        ],
    )
    def tile_task(u_hbm, ue_hbm, rows_hbm, uidx_v, rows_v, sem):
        base = _worker_id() * _BPW
        _gather_rows(u_hbm, ue_hbm, rows_hbm, uidx_v, rows_v, sem, base)

    return tile_task(u, user_emb)


def _prod_gather_dot(p, prod_emb, u_rows):
    @functools.partial(
        pl.kernel,
        mesh=_mesh(),
        out_type=jax.ShapeDtypeStruct((_BATCH,), jnp.float32),
        compiler_params=pltpu.CompilerParams(use_tc_tiling_on_sc=False),
        scratch_types=[
            pltpu.VMEM((_NCH, _CHUNK), jnp.int32),
            pltpu.VMEM((_BPW, _DIM), jnp.float32),
            pltpu.VMEM((_BPW, _DIM), jnp.float32),
            pltpu.VMEM((_BPW,), jnp.float32),
            pltpu.SemaphoreType.DMA,
        ],
    )
    def tile_task(p_hbm, pe_hbm, urows_hbm, out_hbm,
                  pidx_v, prows_v, urows_v, out_v, sem):
        base = _worker_id() * _BPW
        for j in range(_NCH):
            pltpu.sync_copy(p_hbm.at[pl.ds(base + j * _CHUNK, _CHUNK)],
                            pidx_v.at[j])
        copies = [
            pltpu.async_copy(pe_hbm.at[pidx_v.at[j]],
                             prows_v.at[pl.ds(j * _CHUNK, _CHUNK)], sem)
            for j in range(_NCH)
        ]
        copies.append(
            pltpu.async_copy(urows_hbm.at[pl.ds(base, _BPW)], urows_v, sem))
        for c in copies:
            c.wait()

        iota16 = lax.iota(jnp.int32, 16)
        dnums = lax.GatherDimensionNumbers(
            offset_dims=(), collapsed_slice_dims=(0,), start_index_map=(0,))

        def lane_perm(x, idx):
            return lax.gather(
                x, idx[:, None], dimension_numbers=dnums, slice_sizes=(1,),
                mode=lax.GatherScatterMode.PROMISE_IN_BOUNDS)

        def block(t, carry):
            acc = jnp.zeros((16,), jnp.float32)
            for l in range(16):
                r = t * 16 + l
                prod = jnp.zeros((16,), jnp.float32)
                for k in range(_DIM // 16):
                    prod = prod + (urows_v[r, pl.ds(k * 16, 16)]
                                   * prows_v[r, pl.ds(k * 16, 16)])
                for sh in (8, 4, 2, 1):
                    prod = prod + lane_perm(prod, iota16 ^ sh)
                acc = jnp.where(iota16 == l, prod, acc)
            out_v[pl.ds(t * 16, 16)] = 1.0 / (1.0 + jnp.exp(-acc))
            return carry

        lax.fori_loop(0, _BPW // 16, block, 0)
        pltpu.sync_copy(out_v, out_hbm.at[pl.ds(base, _BPW)])

    return tile_task(p, prod_emb, u_rows)


def kernel(u, p, user_emb, prod_emb):
    u_rows = _user_gather(u, user_emb)
    return _prod_gather_dot(p, prod_emb, u_rows)


# speedup vs baseline: 1.5041x; 1.5028x over previous
import functools

import jax
import jax.numpy as jnp
from jax import lax
from jax.experimental import pallas as pl
from jax.experimental.pallas import tpu as pltpu
from jax.experimental.pallas import tpu_sc as plsc

_BATCH = 16384
_DIM = 64
_GRP = 16
_SLOTS = 32


def _two_tower_sc(u, p, user_emb, prod_emb):
    info = plsc.get_sparse_core_info()
    nw = info.num_cores * info.num_subcores
    b_per_w = _BATCH // nw
    n_grp = b_per_w // _GRP
    mesh = plsc.VectorSubcoreMesh(core_axis_name="c", subcore_axis_name="s")

    @functools.partial(
        pl.kernel,
        mesh=mesh,
        out_type=jax.ShapeDtypeStruct((_BATCH,), jnp.float32),
        compiler_params=pltpu.CompilerParams(use_tc_tiling_on_sc=True),
        scratch_types=[
            pltpu.VMEM((b_per_w + 16,), jnp.int32),
            pltpu.VMEM((b_per_w + 16,), jnp.int32),
            pltpu.VMEM((_SLOTS, 8, _DIM), jnp.float32),
            pltpu.VMEM((_SLOTS, 8, _DIM), jnp.float32),
            pltpu.VMEM((b_per_w,), jnp.float32),
            pltpu.SemaphoreType.DMA,
            pltpu.SemaphoreType.DMA,
        ],
    )
    def tile_task(u_hbm, p_hbm, ue_hbm, pe_hbm, out_hbm,
                  uidx_v, pidx_v, ublk_v, pblk_v, out_v, usem, psem):
        wid = lax.axis_index("s") * info.num_cores + lax.axis_index("c")
        base = wid * b_per_w

        pltpu.sync_copy(u_hbm.at[pl.ds(base, b_per_w)],
                        uidx_v.at[pl.ds(0, b_per_w)])
        pltpu.sync_copy(p_hbm.at[pl.ds(base, b_per_w)],
                        pidx_v.at[pl.ds(0, b_per_w)])

        iota16 = lax.iota(jnp.int32, 16)
        dnums = lax.GatherDimensionNumbers(
            offset_dims=(), collapsed_slice_dims=(0,), start_index_map=(0,))

        def lane_perm(x, idx):
            return lax.gather(
                x, idx[:, None], dimension_numbers=dnums, slice_sizes=(1,),
                mode=lax.GatherScatterMode.PROMISE_IN_BOUNDS)

        def fire(b):
            sl = b & (_SLOTS - 1)
            ru = uidx_v[pl.ds(b, 16)][0]
            rp = pidx_v[pl.ds(b, 16)][0]
            u0 = pl.multiple_of((ru >> 3) * 8, 8)
            p0 = pl.multiple_of((rp >> 3) * 8, 8)
            pltpu.async_copy(ue_hbm.at[pl.ds(u0, 8)], ublk_v.at[sl], usem)
            pltpu.async_copy(pe_hbm.at[pl.ds(p0, 8)], pblk_v.at[sl], psem)

        def drain():
            pltpu.make_async_copy(
                ue_hbm.at[pl.ds(0, 8)], ublk_v.at[0], usem).wait()
            pltpu.make_async_copy(
                pe_hbm.at[pl.ds(0, 8)], pblk_v.at[0], psem).wait()

        def consume_group(t):
            acc = jnp.zeros((16,), jnp.float32)
            for l in range(16):
                b = t * _GRP + l
                sl = b & (_SLOTS - 1)
                drain()
                ru = uidx_v[pl.ds(b, 16)][0]
                rp = pidx_v[pl.ds(b, 16)][0]
                rmu = ru & 7
                rmp = rp & 7
                prod = jnp.zeros((16,), jnp.float32)
                for k in range(_DIM // 16):
                    prod = prod + (ublk_v[sl, rmu, pl.ds(k * 16, 16)]
                                   * pblk_v[sl, rmp, pl.ds(k * 16, 16)])
                for sh in (8, 4, 2, 1):
                    prod = prod + lane_perm(prod, iota16 ^ sh)
                acc = jnp.where(iota16 == l, prod, acc)
            out_v[pl.ds(t * _GRP, 16)] = 1.0 / (1.0 + jnp.exp(-acc))

        def fire_group(t):
            for l in range(16):
                fire(t * _GRP + l)

        fire_group(0)

        def steady(t, carry):
            fire_group(t + 1)
            consume_group(t)
            return carry

        lax.fori_loop(0, n_grp - 1, steady, 0)
        consume_group(n_grp - 1)

        pltpu.sync_copy(out_v, out_hbm.at[pl.ds(base, b_per_w)])

    return tile_task(u, p, user_emb, prod_emb)


def kernel(u, p, user_emb, prod_emb):
    return _two_tower_sc(u, p, user_emb, prod_emb)
